# baseline (speedup 1.0000x reference)
import functools

import jax
import jax.numpy as jnp
from jax.experimental import pallas as pl
from jax.experimental.pallas import tpu as pltpu

_LANE = 128
_VMEM_LIMIT = 60 * 1024 * 1024


def _round_up(x, m):
    return ((x + m - 1) // m) * m


def _dist_kernel(e_ref, o_ref, ebf_ref, sqc_ref, sqr_ref, *, tm, tn, nsi, nj):
    c = pl.program_id(0)
    s = pl.program_id(1)
    j = pl.program_id(2)
    jj = jax.lax.rem(j + c, nj)

    @pl.when(s == 0)
    def _cast_tile():
        rs = jj * tn
        et = e_ref[pl.ds(rs, tn), :]
        ebf_ref[pl.ds(rs, tn), :] = et.astype(jnp.bfloat16)
        sqt = jnp.sum(et * et, axis=1, keepdims=True)
        sqc_ref[pl.ds(rs, tn), :] = sqt
        sqr_ref[:, pl.ds(rs, tn)] = jnp.transpose(sqt, (1, 0))

    i = c * nsi + s
    ei = ebf_ref[pl.ds(i * tm, tm), :]
    ej = ebf_ref[pl.ds(jj * tn, tn), :]
    gram = jax.lax.dot_general(
        ei,
        ej,
        dimension_numbers=(((1,), (1,)), ((), ())),
        preferred_element_type=jnp.float32,
    )
    o_ref[...] = (sqc_ref[pl.ds(i * tm, tm), :]
                  + sqr_ref[:, pl.ds(jj * tn, tn)] - 2.0 * gram)


def kernel(embeddings, labels):
    n, d = embeddings.shape
    d_pad = _round_up(d, _LANE)
    if n > 1024:
        tm, nj = 512, 2
        n_pad = _round_up(n, 2048)
    else:
        tm, nj = 256, 1
        n_pad = _round_up(n, 512)
    nsi = n_pad // tm // 2
    tn = n_pad // nj

    e32 = embeddings.astype(jnp.float32)
    if (n_pad, d_pad) == (n, d):
        e_pad = e32
    else:
        e_pad = jnp.zeros((n_pad, d_pad), jnp.float32).at[:n, :d].set(e32)

    dist = pl.pallas_call(
        functools.partial(_dist_kernel, tm=tm, tn=tn, nsi=nsi, nj=nj),
        out_shape=jax.ShapeDtypeStruct((n_pad, n_pad), jnp.float32),
        grid=(2, nsi, nj),
        in_specs=[
            pl.BlockSpec((n_pad, d_pad), lambda c, s, j: (0, 0)),
        ],
        out_specs=pl.BlockSpec(
            (tm, tn), lambda c, s, j: (c * nsi + s, (j + c) % nj)),
        scratch_shapes=[
            pltpu.VMEM((n_pad, d_pad), jnp.bfloat16),
            pltpu.VMEM((n_pad, 1), jnp.float32),
            pltpu.VMEM((1, n_pad), jnp.float32),
        ],
        compiler_params=pltpu.CompilerParams(
            dimension_semantics=("parallel", "arbitrary", "arbitrary"),
            vmem_limit_bytes=_VMEM_LIMIT,
        ),
    )(e_pad)
    return dist[:n, :n]


# speedup vs baseline: 1.0444x; 1.0444x over previous
import functools

import jax
import jax.numpy as jnp
from jax.experimental import pallas as pl
from jax.experimental.pallas import tpu as pltpu

_LANE = 128
_VMEM_LIMIT = 60 * 1024 * 1024


def _round_up(x, m):
    return ((x + m - 1) // m) * m


def _dist_kernel(e_ref, o_ref, ebf_ref, sqc_ref, sqr_ref, *, tm, tn, nsi, nj):
    c = pl.program_id(0)
    s = pl.program_id(1)
    j = pl.program_id(2)
    jj = jax.lax.rem(j + c, nj)

    @pl.when(s == 0)
    def _cast_tile():
        rs = jj * tn
        et = e_ref[pl.ds(rs, tn), :]
        ebf_ref[pl.ds(rs, tn), :] = et.astype(jnp.bfloat16)
        sqt = jnp.sum(et * et, axis=1, keepdims=True)
        sqc_ref[pl.ds(rs, tn), :] = sqt
        sqr_ref[:, pl.ds(rs, tn)] = jnp.transpose(sqt, (1, 0))

    i = c * nsi + s
    ei = ebf_ref[pl.ds(i * tm, tm), :]
    ej = ebf_ref[pl.ds(jj * tn, tn), :]
    gram = jax.lax.dot_general(
        ei,
        ej,
        dimension_numbers=(((1,), (1,)), ((), ())),
        preferred_element_type=jnp.float32,
    )
    o_ref[...] = (sqc_ref[pl.ds(i * tm, tm), :]
                  + sqr_ref[:, pl.ds(jj * tn, tn)] - 2.0 * gram)


def kernel(embeddings, labels):
    n, d = embeddings.shape
    d_pad = _round_up(d, _LANE)
    if n > 1024:
        tm, nj = 512, 1
        n_pad = _round_up(n, 1024)
    else:
        tm, nj = 256, 1
        n_pad = _round_up(n, 512)
    nsi = n_pad // tm // 2
    tn = n_pad // nj

    e32 = embeddings.astype(jnp.float32)
    if (n_pad, d_pad) == (n, d):
        e_pad = e32
    else:
        e_pad = jnp.zeros((n_pad, d_pad), jnp.float32).at[:n, :d].set(e32)

    dist = pl.pallas_call(
        functools.partial(_dist_kernel, tm=tm, tn=tn, nsi=nsi, nj=nj),
        out_shape=jax.ShapeDtypeStruct((n_pad, n_pad), jnp.float32),
        grid=(2, nsi, nj),
        in_specs=[
            pl.BlockSpec((n_pad, d_pad), lambda c, s, j: (0, 0)),
        ],
        out_specs=pl.BlockSpec(
            (tm, tn), lambda c, s, j: (c * nsi + s, (j + c) % nj)),
        scratch_shapes=[
            pltpu.VMEM((n_pad, d_pad), jnp.bfloat16),
            pltpu.VMEM((n_pad, 1), jnp.float32),
            pltpu.VMEM((1, n_pad), jnp.float32),
        ],
        compiler_params=pltpu.CompilerParams(
            dimension_semantics=("parallel", "arbitrary", "arbitrary"),
            vmem_limit_bytes=_VMEM_LIMIT,
        ),
    )(e_pad)
    return dist[:n, :n]
